# baseline (speedup 1.0000x reference)
import jax
import jax.numpy as jnp
from jax import lax
from jax.experimental import pallas as pl
from jax.experimental.pallas import tpu as pltpu
from jax.experimental.pallas import tpu_sc as plsc

L = 16
NBUF = 2
SH = 8


def kernel(x, pos_table, act_table):
    bs, n, c = x.shape
    np1 = pos_table.shape[0]
    na = act_table.shape[0]
    nr = np1 + na
    ns = np1 + SH - 1
    ntl = na - SH + 1
    assert np1 == n + 1 and act_table.shape[1] == c and c % L == 0
    nv = c // L
    nap = -(-(na + 1) // SH) * SH

    mesh = plsc.VectorSubcoreMesh(core_axis_name="c", subcore_axis_name="s")
    nw = mesh.num_cores * mesh.num_subcores
    assert bs % nw == 0
    nb = bs // nw

    def body(x_hbm, pos_hbm, posr_hbm, act_hbm, out_hbm,
             pos_v, x0, x1, s0, act_sh, osh,
             si0, si1, so0, sd0, sa):
        xbufs = [x0, x1]
        slabs = [s0, s0]
        sin = [si0, si1]
        wid = lax.axis_index("s") * mesh.num_cores + lax.axis_index("c")
        sid = lax.axis_index("s")
        base = wid * nb

        pltpu.sync_copy(posr_hbm, pos_v)
        for p in range(NBUF):
            pltpu.sync_copy(pos_hbm.at[pl.ds(0, SH)],
                            slabs[p].at[pl.ds(0, SH)])

        @pl.when(sid == 0)
        def _():
            pltpu.sync_copy(act_hbm, act_sh)
        plsc.subcore_barrier()
        for p in range(NBUF):
            pltpu.sync_copy(act_sh.at[pl.ds(0, SH)],
                            slabs[p].at[pl.ds(np1 - 1, SH)])

        def in_desc(p, i):
            return pltpu.make_async_copy(
                x_hbm.at[base + i], xbufs[p], sin[p])

        def out_desc(p, i):
            return pltpu.make_async_copy(s0, osh.at[sid], so0)

        def out_dma(p, i):
            return pltpu.make_async_copy(
                osh.at[sid], out_hbm.at[base + i, pl.ds(0, ns)], sd0)

        def tail_desc(i):
            return pltpu.make_async_copy(
                act_sh.at[pl.ds(SH, ntl)],
                out_hbm.at[base + i, pl.ds(ns, ntl)], sa)

        def step(k, p):
            tail_desc(k).start()
            in_desc(p, k).wait()

            xbuf = xbufs[p]
            slab = slabs[p]

            @plsc.parallel_loop(0, n, unroll=4)
            def _(j):
                for jj in range(nv):
                    s = pl.ds(jj * L, L)
                    slab[j + 1, s] = xbuf[j, s] + pos_v[j, s]

            if isinstance(k, int):
                if k >= 1:
                    out_dma(p, k - 1).wait()
            else:
                @pl.when(k >= 1)
                def _():
                    out_dma(p, k - 1).wait()
            out_desc(p, k).start()
            if isinstance(k, int):
                if k + NBUF < nb:
                    in_desc(p, k + NBUF).start()
            else:
                @pl.when(k + NBUF < nb)
                def _():
                    in_desc(p, k + NBUF).start()
            out_desc(p, k).wait()
            out_dma(p, k).start()

        for p in range(NBUF):
            in_desc(p, p).start()

        nloop = (nb // NBUF) * NBUF

        @pl.loop(0, nloop, step=NBUF)
        def _(g):
            for p in range(NBUF):
                step(g + p, p)

        for k in range(nloop, nb):
            step(k, k % NBUF)

        out_dma(0, nb - 1).wait()
        for i in range(nb):
            tail_desc(i).wait()

    call = pl.kernel(
        body,
        out_type=jax.ShapeDtypeStruct((bs, nr, c), jnp.float32),
        mesh=mesh,
        scratch_types=[
            pltpu.VMEM((n, c), jnp.float32),
            pltpu.VMEM((n, c), jnp.float32),
            pltpu.VMEM((n, c), jnp.float32),
            pltpu.VMEM((ns, c), jnp.float32),
            pltpu.VMEM_SHARED((nap, c), jnp.float32),
            pltpu.VMEM_SHARED((16, ns, c), jnp.float32),
        ] + [pltpu.SemaphoreType.DMA] * 5,
        compiler_params=pltpu.CompilerParams(use_tc_tiling_on_sc=True),
    )

    act_pad = jnp.pad(act_table, ((1, nap - na - 1), (0, 0)))
    return call(x, pos_table, pos_table[1:], act_pad)


# speedup vs baseline: 1.0286x; 1.0286x over previous
import jax
import jax.numpy as jnp
from jax import lax
from jax.experimental import pallas as pl
from jax.experimental.pallas import tpu as pltpu
from jax.experimental.pallas import tpu_sc as plsc

L = 16
NBUF = 2
SH = 8


def kernel(x, pos_table, act_table):
    bs, n, c = x.shape
    np1 = pos_table.shape[0]
    na = act_table.shape[0]
    nr = np1 + na
    ns = np1 + SH - 1
    ntl = na - SH + 1
    assert np1 == n + 1 and act_table.shape[1] == c and c % L == 0
    nv = c // L
    nap = -(-(na + 1) // SH) * SH

    mesh = plsc.VectorSubcoreMesh(core_axis_name="c", subcore_axis_name="s")
    nw = mesh.num_cores * mesh.num_subcores
    assert bs % nw == 0
    nb = bs // nw

    def body(x_hbm, pos_hbm, posr_hbm, act_hbm, out_hbm,
             pos_v, x0, x1, s0, s1, act_sh,
             si0, si1, so0, so1, sa):
        xbufs = [x0, x1]
        slabs = [s0, s1]
        sin = [si0, si1]
        sout = [so0, so1]
        wid = lax.axis_index("s") * mesh.num_cores + lax.axis_index("c")
        sid = lax.axis_index("s")
        base = wid * nb

        pltpu.sync_copy(posr_hbm, pos_v)
        for p in range(NBUF):
            pltpu.sync_copy(pos_hbm.at[pl.ds(0, SH)],
                            slabs[p].at[pl.ds(0, SH)])

        @pl.when(sid == 0)
        def _():
            pltpu.sync_copy(act_hbm, act_sh)
        plsc.subcore_barrier()
        for p in range(NBUF):
            pltpu.sync_copy(act_sh.at[pl.ds(0, SH)],
                            slabs[p].at[pl.ds(np1 - 1, SH)])

        def in_desc(p, i):
            return pltpu.make_async_copy(
                x_hbm.at[base + i], xbufs[p], sin[p])

        def out_desc(p, i):
            return pltpu.make_async_copy(
                slabs[p], out_hbm.at[base + i, pl.ds(0, ns)], sout[p])

        def tail_desc(i):
            return pltpu.make_async_copy(
                act_sh.at[pl.ds(SH, ntl)],
                out_hbm.at[base + i, pl.ds(ns, ntl)], sa)

        def step(k, p):
            tail_desc(k).start()
            in_desc(p, k).wait()
            if isinstance(k, int):
                if k >= NBUF:
                    out_desc(p, k - NBUF).wait()
            else:
                @pl.when(k >= NBUF)
                def _():
                    out_desc(p, k - NBUF).wait()

            xbuf = xbufs[p]
            slab = slabs[p]

            @plsc.parallel_loop(0, n, unroll=4)
            def _(j):
                for jj in range(nv):
                    s = pl.ds(jj * L, L)
                    slab[j + 1, s] = xbuf[j, s] + pos_v[j, s]

            if isinstance(k, int):
                if k + NBUF < nb:
                    in_desc(p, k + NBUF).start()
            else:
                @pl.when(k + NBUF < nb)
                def _():
                    in_desc(p, k + NBUF).start()
            out_desc(p, k).start()

        for p in range(NBUF):
            in_desc(p, p).start()

        nloop = (nb // NBUF) * NBUF

        @pl.loop(0, nloop, step=NBUF)
        def _(g):
            for p in range(NBUF):
                step(g + p, p)

        for k in range(nloop, nb):
            step(k, k % NBUF)

        for p in range(NBUF):
            out_desc(p, nb - NBUF + p).wait()
        for i in range(nb):
            tail_desc(i).wait()

    call = pl.kernel(
        body,
        out_type=jax.ShapeDtypeStruct((bs, nr, c), jnp.float32),
        mesh=mesh,
        scratch_types=[
            pltpu.VMEM((n, c), jnp.float32),
            pltpu.VMEM((n, c), jnp.float32),
            pltpu.VMEM((n, c), jnp.float32),
            pltpu.VMEM((ns, c), jnp.float32),
            pltpu.VMEM((ns, c), jnp.float32),
            pltpu.VMEM_SHARED((nap, c), jnp.float32),
        ] + [pltpu.SemaphoreType.DMA] * (2 * NBUF + 1),
        compiler_params=pltpu.CompilerParams(use_tc_tiling_on_sc=True),
    )

    act_pad = jnp.pad(act_table, ((1, nap - na - 1), (0, 0)))
    return call(x, pos_table, pos_table[1:], act_pad)
